# baseline (speedup 1.0000x reference)
import functools

import jax
import jax.numpy as jnp
from jax import lax
from jax.experimental import pallas as pl
from jax.experimental.pallas import tpu as pltpu
from jax.experimental.pallas import tpu_sc as plsc

E = 8
K = 2
CAP_FACTOR = 1.5

_NC = 2
_NS = 16
_NW = _NC * _NS


def _router_kernel(x_ref, wrt_ref, d_ref, c_ref, *, capacity, trash):
    x = x_ref[...]
    wrt = wrt_ref[...]
    T = x.shape[0]
    logits = jnp.dot(x, wrt, preferred_element_type=jnp.float32)
    m = jnp.max(logits, axis=-1, keepdims=True)
    ex = jnp.exp(logits - m)
    probs = ex / jnp.sum(ex, axis=-1, keepdims=True)

    lane = jax.lax.broadcasted_iota(jnp.int32, probs.shape, 1)
    p1 = jnp.max(probs, axis=-1, keepdims=True)
    a1 = jnp.min(jnp.where(probs == p1, lane, E), axis=-1, keepdims=True)
    oh1 = (lane == a1).astype(jnp.float32)
    probs2 = jnp.where(lane == a1, -jnp.inf, probs)
    p2 = jnp.max(probs2, axis=-1, keepdims=True)
    a2 = jnp.min(jnp.where(probs2 == p2, lane, E), axis=-1, keepdims=True)
    oh2 = (lane == a2).astype(jnp.float32)

    s = p1 + p2
    p1n = p1 / s
    p2n = p2 / s

    ohsum = oh1 + oh2
    inc = ohsum
    shift = 1
    while shift < T:
        shifted = jnp.concatenate(
            [jnp.zeros((shift, E), jnp.float32), inc[: T - shift]], axis=0)
        inc = inc + shifted
        shift *= 2
    excl = inc - ohsum

    r1 = jnp.sum(oh1 * excl, axis=-1, keepdims=True).astype(jnp.int32)
    r2 = jnp.sum(oh2 * excl, axis=-1, keepdims=True).astype(jnp.int32)

    keep1 = r1 < capacity
    keep2 = r2 < capacity

    d1 = jnp.where(keep1, a1 * capacity + r1, trash)
    d2 = jnp.where(keep2, a2 * capacity + r2, trash)
    c1 = jnp.where(keep1, p1n, 0.0)
    c2 = jnp.where(keep2, p2n, 0.0)

    d_ref[...] = jnp.concatenate([d1, d2], axis=1)
    c_ref[...] = jnp.concatenate([c1, c2], axis=1)


def _sc_dispatch_gather_body(x_hbm, d1_hbm, d2_hbm, c1_hbm, c2_hbm,
                             xg_hbm, w_hbm,
                             dv_v, cv_v, tok_v, w_v, idx_v, rows_v, tok_sh,
                             sem, *, n_slots, n_tok, rows_per_w):
    cid = lax.axis_index("c")
    sid = lax.axis_index("s")

    @pl.when(sid == 0)
    def _():
        zi = jnp.zeros((16,), jnp.int32)
        zf = jnp.zeros((16,), jnp.float32)

        def zero_body(i, _):
            tok_v[pl.ds(i * 16, 16)] = zi
            w_v[pl.ds(i * 16, 16)] = zf
            return 0

        lax.fori_loop(0, n_slots // 16, zero_body, 0)

        iota16 = lax.iota(jnp.int32, 16)

        def scatter_from(d_hbm, c_hbm):
            pltpu.sync_copy(d_hbm, dv_v)
            pltpu.sync_copy(c_hbm, cv_v)

            def body(i, _):
                dv = dv_v[pl.ds(i * 16, 16)]
                cv = cv_v[pl.ds(i * 16, 16)]
                tokv = iota16 + i * 16
                plsc.store_scatter(tok_v, [dv], tokv)
                plsc.store_scatter(w_v, [dv], cv)
                return 0

            lax.fori_loop(0, n_tok // 16, body, 0)

        scatter_from(d1_hbm, c1_hbm)
        scatter_from(d2_hbm, c2_hbm)

        pltpu.sync_copy(tok_v, tok_sh)

        @pl.when(cid == 0)
        def _():
            pltpu.sync_copy(w_v, w_hbm)

    plsc.subcore_barrier()

    base = (cid * _NS + sid) * rows_per_w
    pltpu.sync_copy(tok_sh.at[pl.ds(base, rows_per_w)], idx_v)
    pltpu.async_copy(x_hbm.at[idx_v], rows_v, sem).wait()
    pltpu.sync_copy(rows_v, xg_hbm.at[pl.ds(base, rows_per_w)])


def _ffn_kernel(xg_ref, w1_ref, b1_ref, w2_ref, b2_ref, wc_ref, out_ref):
    e = pl.program_id(0)

    @pl.when(e < E)
    def _():
        xg = xg_ref[...]
        h = jnp.dot(xg, w1_ref[0], preferred_element_type=jnp.float32)
        h = h + b1_ref[0]
        h = 0.5 * h * (1.0 + jax.lax.erf(h * 0.7071067811865476))
        o = jnp.dot(h, w2_ref[0], preferred_element_type=jnp.float32)
        o = o + b2_ref[0]
        out_ref[...] = o * wc_ref[:, 0:1]

    @pl.when(e >= E)
    def _():
        out_ref[...] = jnp.zeros_like(out_ref)


def _sc_combine_body(yw_hbm, d1_hbm, d2_hbm, o12_hbm,
                     i1_v, i2_v, r1_v, r2_v, sem1, sem2,
                     *, tok_per_w, chunk):
    cid = lax.axis_index("c")
    sid = lax.axis_index("s")
    gb = (cid * _NS + sid) * tok_per_w
    n_chunks = tok_per_w // chunk

    def body(ci, _):
        tb = gb + ci * chunk
        pltpu.sync_copy(d1_hbm.at[pl.ds(tb, chunk)], i1_v)
        pltpu.sync_copy(d2_hbm.at[pl.ds(tb, chunk)], i2_v)
        cp1 = pltpu.async_copy(yw_hbm.at[i1_v], r1_v, sem1)
        cp2 = pltpu.async_copy(yw_hbm.at[i2_v], r2_v, sem2)
        cp1.wait()
        pltpu.sync_copy(r1_v, o12_hbm.at[0, pl.ds(tb, chunk)])
        cp2.wait()
        pltpu.sync_copy(r2_v, o12_hbm.at[1, pl.ds(tb, chunk)])
        return 0

    lax.fori_loop(0, n_chunks, body, 0)


def _add_kernel(o12_ref, out_ref):
    out_ref[...] = o12_ref[0] + o12_ref[1]


@jax.jit
def _moe(x, Wr, w1, b1, w2, b2):
    B, S, D = x.shape
    T = B * S
    H = w1.shape[-1]
    O = w2.shape[-1]
    capacity = int((T / E) * CAP_FACTOR)
    n_rows = E * capacity
    n_slots = n_rows + 16
    trash = n_rows

    xt = x.reshape(T, D)

    d, c = pl.pallas_call(
        functools.partial(_router_kernel, capacity=capacity, trash=trash),
        out_shape=(
            jax.ShapeDtypeStruct((T, 2), jnp.int32),
            jax.ShapeDtypeStruct((T, 2), jnp.float32),
        ),
    )(xt, Wr.T)
    d1 = d[:, 0]
    d2 = d[:, 1]
    c1 = c[:, 0]
    c2 = c[:, 1]

    mesh = plsc.VectorSubcoreMesh(core_axis_name="c", subcore_axis_name="s")
    sc_params = pltpu.CompilerParams(needs_layout_passes=False)

    rows_per_w = n_rows // _NW
    xg, w_pos = pl.kernel(
        functools.partial(_sc_dispatch_gather_body, n_slots=n_slots,
                          n_tok=T, rows_per_w=rows_per_w),
        mesh=mesh,
        out_type=(
            jax.ShapeDtypeStruct((n_rows, D), jnp.float32),
            jax.ShapeDtypeStruct((n_slots,), jnp.float32),
        ),
        scratch_types=[
            pltpu.VMEM((T,), jnp.int32),
            pltpu.VMEM((T,), jnp.float32),
            pltpu.VMEM((n_slots,), jnp.int32),
            pltpu.VMEM((n_slots,), jnp.float32),
            pltpu.VMEM((rows_per_w,), jnp.int32),
            pltpu.VMEM((rows_per_w, D), jnp.float32),
            pltpu.VMEM_SHARED((n_slots,), jnp.int32),
            pltpu.SemaphoreType.DMA,
        ],
        compiler_params=sc_params,
    )(xt, d1, d2, c1, c2)

    wcol = jnp.broadcast_to(w_pos[:n_rows, None], (n_rows, 128))
    yw = pl.pallas_call(
        _ffn_kernel,
        grid=(E + 1,),
        in_specs=[
            pl.BlockSpec((capacity, D), lambda e: (jnp.where(e >= E, 0, e), 0)),
            pl.BlockSpec((1, D, H), lambda e: (jnp.where(e >= E, 0, e), 0, 0)),
            pl.BlockSpec((1, 1, H), lambda e: (jnp.where(e >= E, 0, e), 0, 0)),
            pl.BlockSpec((1, H, O), lambda e: (jnp.where(e >= E, 0, e), 0, 0)),
            pl.BlockSpec((1, 1, O), lambda e: (jnp.where(e >= E, 0, e), 0, 0)),
            pl.BlockSpec((capacity, 128), lambda e: (jnp.where(e >= E, 0, e), 0)),
        ],
        out_specs=pl.BlockSpec((capacity, O), lambda e: (e, 0)),
        out_shape=jax.ShapeDtypeStruct((n_rows + capacity, O), jnp.float32),
        compiler_params=pltpu.CompilerParams(
            dimension_semantics=("arbitrary",),
        ),
    )(xg, w1, b1.reshape(E, 1, H), w2, b2.reshape(E, 1, O), wcol)

    tok_per_w = T // _NW
    chunk = 32
    o12 = pl.kernel(
        functools.partial(_sc_combine_body, tok_per_w=tok_per_w, chunk=chunk),
        mesh=mesh,
        out_type=jax.ShapeDtypeStruct((2, T, O), jnp.float32),
        scratch_types=[
            pltpu.VMEM((chunk,), jnp.int32),
            pltpu.VMEM((chunk,), jnp.int32),
            pltpu.VMEM((chunk, O), jnp.float32),
            pltpu.VMEM((chunk, O), jnp.float32),
            pltpu.SemaphoreType.DMA,
            pltpu.SemaphoreType.DMA,
        ],
        compiler_params=sc_params,
    )(yw, d1, d2)

    out = pl.pallas_call(
        _add_kernel,
        out_shape=jax.ShapeDtypeStruct((T, O), jnp.float32),
    )(o12)

    return out.reshape(B, S, O)


def kernel(x, Wr, w1, b1, w2, b2):
    return _moe(x, Wr, w1, b1, w2, b2)


# speedup vs baseline: 1.0175x; 1.0175x over previous
import functools

import jax
import jax.numpy as jnp
from jax import lax
from jax.experimental import pallas as pl
from jax.experimental.pallas import tpu as pltpu
from jax.experimental.pallas import tpu_sc as plsc

E = 8
K = 2
CAP_FACTOR = 1.5

_NC = 2
_NS = 16
_NW = _NC * _NS


def _router_kernel(x_ref, wrt_ref, d_ref, c_ref, *, capacity, trash):
    x = x_ref[...]
    wrt = wrt_ref[...]
    T = x.shape[0]
    logits = jnp.dot(x, wrt, preferred_element_type=jnp.float32)
    m = jnp.max(logits, axis=-1, keepdims=True)
    ex = jnp.exp(logits - m)
    probs = ex / jnp.sum(ex, axis=-1, keepdims=True)

    lane = jax.lax.broadcasted_iota(jnp.int32, probs.shape, 1)
    p1 = jnp.max(probs, axis=-1, keepdims=True)
    a1 = jnp.min(jnp.where(probs == p1, lane, E), axis=-1, keepdims=True)
    oh1 = (lane == a1).astype(jnp.float32)
    probs2 = jnp.where(lane == a1, -jnp.inf, probs)
    p2 = jnp.max(probs2, axis=-1, keepdims=True)
    a2 = jnp.min(jnp.where(probs2 == p2, lane, E), axis=-1, keepdims=True)
    oh2 = (lane == a2).astype(jnp.float32)

    s = p1 + p2
    p1n = p1 / s
    p2n = p2 / s

    ohsum = oh1 + oh2
    inc = ohsum
    shift = 1
    while shift < T:
        shifted = jnp.concatenate(
            [jnp.zeros((shift, E), jnp.float32), inc[: T - shift]], axis=0)
        inc = inc + shifted
        shift *= 2
    excl = inc - ohsum

    r1 = jnp.sum(oh1 * excl, axis=-1, keepdims=True).astype(jnp.int32)
    r2 = jnp.sum(oh2 * excl, axis=-1, keepdims=True).astype(jnp.int32)

    keep1 = r1 < capacity
    keep2 = r2 < capacity

    d1 = jnp.where(keep1, a1 * capacity + r1, trash)
    d2 = jnp.where(keep2, a2 * capacity + r2, trash)
    c1 = jnp.where(keep1, p1n, 0.0)
    c2 = jnp.where(keep2, p2n, 0.0)

    d_ref[...] = jnp.concatenate([d1, d2], axis=1)
    c_ref[...] = jnp.concatenate([c1, c2], axis=1)


def _sc_dispatch_gather_body(x_hbm, d1_hbm, d2_hbm, c1_hbm, c2_hbm,
                             xg_hbm, w_hbm,
                             dv_v, cv_v, tok_v, w_v, idx_v, rows_v, tok_sh,
                             sem, *, n_slots, n_tok, rows_per_w):
    cid = lax.axis_index("c")
    sid = lax.axis_index("s")

    @pl.when(sid == 0)
    def _():
        zi = jnp.zeros((16,), jnp.int32)
        zf = jnp.zeros((16,), jnp.float32)

        def zero_body(i, _):
            tok_v[pl.ds(i * 16, 16)] = zi
            w_v[pl.ds(i * 16, 16)] = zf
            return 0

        lax.fori_loop(0, n_slots // 16, zero_body, 0)

        iota16 = lax.iota(jnp.int32, 16)

        def scatter_from(d_hbm, c_hbm):
            pltpu.sync_copy(d_hbm, dv_v)
            pltpu.sync_copy(c_hbm, cv_v)

            def body(i, _):
                dv = dv_v[pl.ds(i * 16, 16)]
                cv = cv_v[pl.ds(i * 16, 16)]
                tokv = iota16 + i * 16
                plsc.store_scatter(tok_v, [dv], tokv)
                plsc.store_scatter(w_v, [dv], cv)
                return 0

            lax.fori_loop(0, n_tok // 16, body, 0)

        scatter_from(d1_hbm, c1_hbm)
        scatter_from(d2_hbm, c2_hbm)

        pltpu.sync_copy(tok_v, tok_sh)

        @pl.when(cid == 0)
        def _():
            pltpu.sync_copy(w_v, w_hbm)

    plsc.subcore_barrier()

    base = (cid * _NS + sid) * rows_per_w
    pltpu.sync_copy(tok_sh.at[pl.ds(base, rows_per_w)], idx_v)
    pltpu.async_copy(x_hbm.at[idx_v], rows_v, sem).wait()
    pltpu.sync_copy(rows_v, xg_hbm.at[pl.ds(base, rows_per_w)])


def _ffn_kernel(xg_ref, w1_ref, b1_ref, w2_ref, b2_ref, wc_ref, out_ref):
    e = pl.program_id(0)

    @pl.when(e < E)
    def _():
        xg = xg_ref[...]
        h = jnp.dot(xg, w1_ref[0], preferred_element_type=jnp.float32)
        h = h + b1_ref[0]
        h = 0.5 * h * (1.0 + jax.lax.erf(h * 0.7071067811865476))
        o = jnp.dot(h, w2_ref[0], preferred_element_type=jnp.float32)
        o = o + b2_ref[0]
        out_ref[...] = o * wc_ref[:, 0:1]

    @pl.when(e >= E)
    def _():
        out_ref[...] = jnp.zeros_like(out_ref)


def _sc_combine_body(yw_hbm, d1_hbm, d2_hbm, o12_hbm,
                     i1_v, i2_v, r1_v, r2_v, sem1, sem2, sem3, sem4,
                     *, tok_per_w):
    cid = lax.axis_index("c")
    sid = lax.axis_index("s")
    gb = (cid * _NS + sid) * tok_per_w

    cpi1 = pltpu.async_copy(d1_hbm.at[pl.ds(gb, tok_per_w)], i1_v, sem1)
    cpi2 = pltpu.async_copy(d2_hbm.at[pl.ds(gb, tok_per_w)], i2_v, sem2)
    cpi1.wait()
    cp1 = pltpu.async_copy(yw_hbm.at[i1_v], r1_v, sem3)
    cpi2.wait()
    cp2 = pltpu.async_copy(yw_hbm.at[i2_v], r2_v, sem4)
    cp1.wait()
    cpo1 = pltpu.async_copy(r1_v, o12_hbm.at[0, pl.ds(gb, tok_per_w)], sem1)
    cp2.wait()
    cpo2 = pltpu.async_copy(r2_v, o12_hbm.at[1, pl.ds(gb, tok_per_w)], sem2)
    cpo1.wait()
    cpo2.wait()


def _add_kernel(o12_ref, out_ref):
    out_ref[...] = o12_ref[0] + o12_ref[1]


@jax.jit
def _moe(x, Wr, w1, b1, w2, b2):
    B, S, D = x.shape
    T = B * S
    H = w1.shape[-1]
    O = w2.shape[-1]
    capacity = int((T / E) * CAP_FACTOR)
    n_rows = E * capacity
    n_slots = n_rows + 16
    trash = n_rows

    xt = x.reshape(T, D)

    d, c = pl.pallas_call(
        functools.partial(_router_kernel, capacity=capacity, trash=trash),
        out_shape=(
            jax.ShapeDtypeStruct((T, 2), jnp.int32),
            jax.ShapeDtypeStruct((T, 2), jnp.float32),
        ),
    )(xt, Wr.T)
    d1 = d[:, 0]
    d2 = d[:, 1]
    c1 = c[:, 0]
    c2 = c[:, 1]

    mesh = plsc.VectorSubcoreMesh(core_axis_name="c", subcore_axis_name="s")
    sc_params = pltpu.CompilerParams(needs_layout_passes=False)

    rows_per_w = n_rows // _NW
    xg, w_pos = pl.kernel(
        functools.partial(_sc_dispatch_gather_body, n_slots=n_slots,
                          n_tok=T, rows_per_w=rows_per_w),
        mesh=mesh,
        out_type=(
            jax.ShapeDtypeStruct((n_rows, D), jnp.float32),
            jax.ShapeDtypeStruct((n_slots,), jnp.float32),
        ),
        scratch_types=[
            pltpu.VMEM((T,), jnp.int32),
            pltpu.VMEM((T,), jnp.float32),
            pltpu.VMEM((n_slots,), jnp.int32),
            pltpu.VMEM((n_slots,), jnp.float32),
            pltpu.VMEM((rows_per_w,), jnp.int32),
            pltpu.VMEM((rows_per_w, D), jnp.float32),
            pltpu.VMEM_SHARED((n_slots,), jnp.int32),
            pltpu.SemaphoreType.DMA,
        ],
        compiler_params=sc_params,
    )(xt, d1, d2, c1, c2)

    wcol = jnp.broadcast_to(w_pos[:n_rows, None], (n_rows, 128))
    yw = pl.pallas_call(
        _ffn_kernel,
        grid=(E + 1,),
        in_specs=[
            pl.BlockSpec((capacity, D), lambda e: (jnp.where(e >= E, 0, e), 0)),
            pl.BlockSpec((1, D, H), lambda e: (jnp.where(e >= E, 0, e), 0, 0)),
            pl.BlockSpec((1, 1, H), lambda e: (jnp.where(e >= E, 0, e), 0, 0)),
            pl.BlockSpec((1, H, O), lambda e: (jnp.where(e >= E, 0, e), 0, 0)),
            pl.BlockSpec((1, 1, O), lambda e: (jnp.where(e >= E, 0, e), 0, 0)),
            pl.BlockSpec((capacity, 128), lambda e: (jnp.where(e >= E, 0, e), 0)),
        ],
        out_specs=pl.BlockSpec((capacity, O), lambda e: (e, 0)),
        out_shape=jax.ShapeDtypeStruct((n_rows + capacity, O), jnp.float32),
        compiler_params=pltpu.CompilerParams(
            dimension_semantics=("arbitrary",),
        ),
    )(xg, w1, b1.reshape(E, 1, H), w2, b2.reshape(E, 1, O), wcol)

    tok_per_w = T // _NW
    o12 = pl.kernel(
        functools.partial(_sc_combine_body, tok_per_w=tok_per_w),
        mesh=mesh,
        out_type=jax.ShapeDtypeStruct((2, T, O), jnp.float32),
        scratch_types=[
            pltpu.VMEM((tok_per_w,), jnp.int32),
            pltpu.VMEM((tok_per_w,), jnp.int32),
            pltpu.VMEM((tok_per_w, O), jnp.float32),
            pltpu.VMEM((tok_per_w, O), jnp.float32),
            pltpu.SemaphoreType.DMA,
            pltpu.SemaphoreType.DMA,
            pltpu.SemaphoreType.DMA,
            pltpu.SemaphoreType.DMA,
        ],
        compiler_params=sc_params,
    )(yw, d1, d2)

    out = pl.pallas_call(
        _add_kernel,
        out_shape=jax.ShapeDtypeStruct((T, O), jnp.float32),
    )(o12)

    return out.reshape(B, S, O)


def kernel(x, Wr, w1, b1, w2, b2):
    return _moe(x, Wr, w1, b1, w2, b2)


# speedup vs baseline: 1.7964x; 1.7656x over previous
import functools

import jax
import jax.numpy as jnp
from jax import lax
from jax.experimental import pallas as pl
from jax.experimental.pallas import tpu as pltpu
from jax.experimental.pallas import tpu_sc as plsc

E = 8
K = 2
CAP_FACTOR = 1.5

_NC = 2
_NS = 16
_NW = _NC * _NS


def _router_kernel(x_ref, wrt_ref, d_ref, c_ref, *, capacity, trash):
    x = x_ref[...]
    wrt = wrt_ref[...]
    T = x.shape[0]
    logits = jnp.dot(x, wrt, preferred_element_type=jnp.float32)
    m = jnp.max(logits, axis=-1, keepdims=True)
    ex = jnp.exp(logits - m)
    probs = ex / jnp.sum(ex, axis=-1, keepdims=True)

    lane = jax.lax.broadcasted_iota(jnp.int32, probs.shape, 1)
    p1 = jnp.max(probs, axis=-1, keepdims=True)
    a1 = jnp.min(jnp.where(probs == p1, lane, E), axis=-1, keepdims=True)
    oh1 = (lane == a1).astype(jnp.float32)
    probs2 = jnp.where(lane == a1, -jnp.inf, probs)
    p2 = jnp.max(probs2, axis=-1, keepdims=True)
    a2 = jnp.min(jnp.where(probs2 == p2, lane, E), axis=-1, keepdims=True)
    oh2 = (lane == a2).astype(jnp.float32)

    s = p1 + p2
    p1n = p1 / s
    p2n = p2 / s

    ohsum = oh1 + oh2
    inc = ohsum
    shift = 1
    while shift < T:
        shifted = jnp.concatenate(
            [jnp.zeros((shift, E), jnp.float32), inc[: T - shift]], axis=0)
        inc = inc + shifted
        shift *= 2
    excl = inc - ohsum

    r1 = jnp.sum(oh1 * excl, axis=-1, keepdims=True).astype(jnp.int32)
    r2 = jnp.sum(oh2 * excl, axis=-1, keepdims=True).astype(jnp.int32)

    keep1 = r1 < capacity
    keep2 = r2 < capacity

    d1 = jnp.where(keep1, a1 * capacity + r1, trash)
    d2 = jnp.where(keep2, a2 * capacity + r2, trash)
    c1 = jnp.where(keep1, p1n, 0.0)
    c2 = jnp.where(keep2, p2n, 0.0)

    d_ref[...] = jnp.concatenate([d1, d2], axis=1)
    c_ref[...] = jnp.concatenate([c1, c2], axis=1)


def _sc_dispatch_gather_body(x_hbm, d1_hbm, d2_hbm, xg_hbm,
                             dv_v, tok_v, idx_v, rows_v, tok_sh,
                             sem, *, n_slots, n_tok, rows_per_w):
    cid = lax.axis_index("c")
    sid = lax.axis_index("s")

    @pl.when(sid == 0)
    def _():
        zi = jnp.zeros((16,), jnp.int32)

        def zero_body(i, _):
            tok_v[pl.ds(i * 16, 16)] = zi
            return 0

        lax.fori_loop(0, n_slots // 16, zero_body, 0)

        iota16 = lax.iota(jnp.int32, 16)

        def scatter_from(d_hbm):
            pltpu.sync_copy(d_hbm, dv_v)

            def body(i, _):
                dv = dv_v[pl.ds(i * 16, 16)]
                tokv = iota16 + i * 16
                plsc.store_scatter(tok_v, [dv], tokv)
                return 0

            lax.fori_loop(0, n_tok // 16, body, 0)

        scatter_from(d1_hbm)
        scatter_from(d2_hbm)

        pltpu.sync_copy(tok_v, tok_sh)

    plsc.subcore_barrier()

    base = (cid * _NS + sid) * rows_per_w
    pltpu.sync_copy(tok_sh.at[pl.ds(base, rows_per_w)], idx_v)
    pltpu.async_copy(x_hbm.at[idx_v], rows_v, sem).wait()
    pltpu.sync_copy(rows_v, xg_hbm.at[pl.ds(base, rows_per_w)])


def _ffn_combine_kernel(xg_ref, w1_ref, b1_ref, w2_ref, b2_ref, d_ref, c_ref,
                        out_ref, *, capacity):
    e = pl.program_id(0)
    xg = xg_ref[...]
    h = jnp.dot(xg, w1_ref[0], preferred_element_type=jnp.float32)
    h = h + b1_ref[0]
    h = 0.5 * h * (1.0 + jax.lax.erf(h * 0.7071067811865476))
    y = jnp.dot(h, w2_ref[0], preferred_element_type=jnp.float32)
    y = y + b2_ref[0]

    T = d_ref.shape[0]
    p_iota = jax.lax.broadcasted_iota(jnp.int32, (T, capacity), 1)
    p_iota = p_iota + e * capacity
    d1 = d_ref[:, 0:1]
    d2 = d_ref[:, 1:2]
    c1 = c_ref[:, 0:1]
    c2 = c_ref[:, 1:2]
    g = jnp.where(p_iota == d1, c1, 0.0) + jnp.where(p_iota == d2, c2, 0.0)
    contrib = jnp.dot(g, y, preferred_element_type=jnp.float32)

    @pl.when(e == 0)
    def _():
        out_ref[...] = contrib

    @pl.when(e != 0)
    def _():
        out_ref[...] = out_ref[...] + contrib


@jax.jit
def _moe(x, Wr, w1, b1, w2, b2):
    B, S, D = x.shape
    T = B * S
    H = w1.shape[-1]
    O = w2.shape[-1]
    capacity = int((T / E) * CAP_FACTOR)
    n_rows = E * capacity
    n_slots = n_rows + 16
    trash = n_rows

    xt = x.reshape(T, D)

    d, c = pl.pallas_call(
        functools.partial(_router_kernel, capacity=capacity, trash=trash),
        out_shape=(
            jax.ShapeDtypeStruct((T, 2), jnp.int32),
            jax.ShapeDtypeStruct((T, 2), jnp.float32),
        ),
    )(xt, Wr.T)
    d1 = d[:, 0]
    d2 = d[:, 1]

    mesh = plsc.VectorSubcoreMesh(core_axis_name="c", subcore_axis_name="s")
    sc_params = pltpu.CompilerParams(needs_layout_passes=False)

    rows_per_w = n_rows // _NW
    xg = pl.kernel(
        functools.partial(_sc_dispatch_gather_body, n_slots=n_slots,
                          n_tok=T, rows_per_w=rows_per_w),
        mesh=mesh,
        out_type=jax.ShapeDtypeStruct((n_rows, D), jnp.float32),
        scratch_types=[
            pltpu.VMEM((T,), jnp.int32),
            pltpu.VMEM((n_slots,), jnp.int32),
            pltpu.VMEM((rows_per_w,), jnp.int32),
            pltpu.VMEM((rows_per_w, D), jnp.float32),
            pltpu.VMEM_SHARED((n_slots,), jnp.int32),
            pltpu.SemaphoreType.DMA,
        ],
        compiler_params=sc_params,
    )(xt, d1, d2)

    out = pl.pallas_call(
        functools.partial(_ffn_combine_kernel, capacity=capacity),
        grid=(E,),
        in_specs=[
            pl.BlockSpec((capacity, D), lambda e: (e, 0)),
            pl.BlockSpec((1, D, H), lambda e: (e, 0, 0)),
            pl.BlockSpec((1, 1, H), lambda e: (e, 0, 0)),
            pl.BlockSpec((1, H, O), lambda e: (e, 0, 0)),
            pl.BlockSpec((1, 1, O), lambda e: (e, 0, 0)),
            pl.BlockSpec((T, 2), lambda e: (0, 0)),
            pl.BlockSpec((T, 2), lambda e: (0, 0)),
        ],
        out_specs=pl.BlockSpec((T, O), lambda e: (0, 0)),
        out_shape=jax.ShapeDtypeStruct((T, O), jnp.float32),
        compiler_params=pltpu.CompilerParams(
            dimension_semantics=("arbitrary",),
        ),
    )(xg, w1, b1.reshape(E, 1, H), w2, b2.reshape(E, 1, O), d, c)

    return out.reshape(B, S, O)


def kernel(x, Wr, w1, b1, w2, b2):
    return _moe(x, Wr, w1, b1, w2, b2)


# speedup vs baseline: 1.9713x; 1.0973x over previous
import functools

import jax
import jax.numpy as jnp
from jax import lax
from jax.experimental import pallas as pl
from jax.experimental.pallas import tpu as pltpu
from jax.experimental.pallas import tpu_sc as plsc

E = 8
K = 2
CAP_FACTOR = 1.5

_NC = 2
_NS = 16
_NW = _NC * _NS


def _router_kernel(x_ref, wr_ref, d_ref, c_ref, *, capacity, trash):
    x = x_ref[...]
    wr = wr_ref[...]
    T = x.shape[0]
    lt = jax.lax.dot_general(
        wr, x, (((1,), (1,)), ((), ())),
        preferred_element_type=jnp.float32)
    m = jnp.max(lt, axis=0, keepdims=True)
    ex = jnp.exp(lt - m)
    probs = ex / jnp.sum(ex, axis=0, keepdims=True)

    erow = jax.lax.broadcasted_iota(jnp.int32, probs.shape, 0)
    p1 = jnp.max(probs, axis=0, keepdims=True)
    a1 = jnp.min(jnp.where(probs == p1, erow, E), axis=0, keepdims=True)
    oh1 = (erow == a1).astype(jnp.float32)
    probs2 = jnp.where(erow == a1, -jnp.inf, probs)
    p2 = jnp.max(probs2, axis=0, keepdims=True)
    a2 = jnp.min(jnp.where(probs2 == p2, erow, E), axis=0, keepdims=True)
    oh2 = (erow == a2).astype(jnp.float32)

    s = p1 + p2
    p1n = p1 / s
    p2n = p2 / s

    ohsum = oh1 + oh2
    inc = ohsum
    shift = 1
    while shift < T:
        shifted = jnp.concatenate(
            [jnp.zeros((E, shift), jnp.float32), inc[:, : T - shift]], axis=1)
        inc = inc + shifted
        shift *= 2
    excl = inc - ohsum

    r1 = jnp.sum(oh1 * excl, axis=0, keepdims=True).astype(jnp.int32)
    r2 = jnp.sum(oh2 * excl, axis=0, keepdims=True).astype(jnp.int32)

    keep1 = r1 < capacity
    keep2 = r2 < capacity

    d1 = jnp.where(keep1, a1 * capacity + r1, trash)
    d2 = jnp.where(keep2, a2 * capacity + r2, trash)
    c1 = jnp.where(keep1, p1n, 0.0)
    c2 = jnp.where(keep2, p2n, 0.0)

    d_ref[...] = jnp.concatenate([d1, d2], axis=0)
    c_ref[...] = jnp.concatenate([c1, c2], axis=0)


def _sc_dispatch_gather_body(x_hbm, d_hbm, c_hbm, xg_hbm, tok_hbm, w_hbm,
                             dv_v, cv_v, tok_v, w_v, idx_v, rows_v, tok_sh,
                             sem, *, n_slots, n_tok, rows_per_w):
    cid = lax.axis_index("c")
    sid = lax.axis_index("s")

    @pl.when(sid == 0)
    def _():
        zi = jnp.zeros((16,), jnp.int32)
        zf = jnp.zeros((16,), jnp.float32)

        def zero_body(i, _):
            tok_v[pl.ds(i * 16, 16)] = zi
            w_v[pl.ds(i * 16, 16)] = zf
            return 0

        lax.fori_loop(0, n_slots // 16, zero_body, 0)

        iota16 = lax.iota(jnp.int32, 16)

        def scatter_from(k):
            pltpu.sync_copy(d_hbm.at[k], dv_v)
            pltpu.sync_copy(c_hbm.at[k], cv_v)

            def body(i, _):
                dv = dv_v[pl.ds(i * 16, 16)]
                cv = cv_v[pl.ds(i * 16, 16)]
                tokv = iota16 + i * 16
                plsc.store_scatter(tok_v, [dv], tokv)
                plsc.store_scatter(w_v, [dv], cv)
                return 0

            lax.fori_loop(0, n_tok // 16, body, 0)

        scatter_from(0)
        scatter_from(1)

        pltpu.sync_copy(tok_v, tok_sh)

        @pl.when(cid == 0)
        def _():
            pltpu.sync_copy(tok_v, tok_hbm)
            pltpu.sync_copy(w_v, w_hbm)

    plsc.subcore_barrier()

    base = (cid * _NS + sid) * rows_per_w
    pltpu.sync_copy(tok_sh.at[pl.ds(base, rows_per_w)], idx_v)
    pltpu.async_copy(x_hbm.at[idx_v], rows_v, sem).wait()
    pltpu.sync_copy(rows_v, xg_hbm.at[pl.ds(base, rows_per_w)])


def _ffn_combine_kernel(xg_ref, w1_ref, b1_ref, w2_ref, b2_ref, tok_ref,
                        w_ref, out_ref, *, capacity):
    e = pl.program_id(0)
    xg = xg_ref[...]
    h = jnp.dot(xg, w1_ref[0], preferred_element_type=jnp.float32)
    h = h + b1_ref[0]
    h = 0.5 * h * (1.0 + jax.lax.erf(h * 0.7071067811865476))
    y = jnp.dot(h, w2_ref[0], preferred_element_type=jnp.float32)
    y = y + b2_ref[0]

    T = out_ref.shape[0]
    tok_row = tok_ref[0]
    w_row = w_ref[0]
    t_col = jax.lax.broadcasted_iota(jnp.int32, (T, capacity), 0)
    g = jnp.where(tok_row == t_col, w_row, 0.0)
    contrib = jnp.dot(g, y, preferred_element_type=jnp.float32)

    @pl.when(e == 0)
    def _():
        out_ref[...] = contrib

    @pl.when(e != 0)
    def _():
        out_ref[...] = out_ref[...] + contrib


@jax.jit
def _moe(x, Wr, w1, b1, w2, b2):
    B, S, D = x.shape
    T = B * S
    H = w1.shape[-1]
    O = w2.shape[-1]
    capacity = int((T / E) * CAP_FACTOR)
    n_rows = E * capacity
    n_slots = n_rows + 16
    trash = n_rows

    xt = x.reshape(T, D)

    d, c = pl.pallas_call(
        functools.partial(_router_kernel, capacity=capacity, trash=trash),
        out_shape=(
            jax.ShapeDtypeStruct((2, T), jnp.int32),
            jax.ShapeDtypeStruct((2, T), jnp.float32),
        ),
    )(xt, Wr)

    mesh = plsc.VectorSubcoreMesh(core_axis_name="c", subcore_axis_name="s")
    sc_params = pltpu.CompilerParams(needs_layout_passes=False)

    rows_per_w = n_rows // _NW
    xg, tok_idx, w_pos = pl.kernel(
        functools.partial(_sc_dispatch_gather_body, n_slots=n_slots,
                          n_tok=T, rows_per_w=rows_per_w),
        mesh=mesh,
        out_type=(
            jax.ShapeDtypeStruct((n_rows, D), jnp.float32),
            jax.ShapeDtypeStruct((n_slots,), jnp.int32),
            jax.ShapeDtypeStruct((n_slots,), jnp.float32),
        ),
        scratch_types=[
            pltpu.VMEM((T,), jnp.int32),
            pltpu.VMEM((T,), jnp.float32),
            pltpu.VMEM((n_slots,), jnp.int32),
            pltpu.VMEM((n_slots,), jnp.float32),
            pltpu.VMEM((rows_per_w,), jnp.int32),
            pltpu.VMEM((rows_per_w, D), jnp.float32),
            pltpu.VMEM_SHARED((n_slots,), jnp.int32),
            pltpu.SemaphoreType.DMA,
        ],
        compiler_params=sc_params,
    )(xt, d, c)

    out = pl.pallas_call(
        functools.partial(_ffn_combine_kernel, capacity=capacity),
        grid=(E,),
        in_specs=[
            pl.BlockSpec((capacity, D), lambda e: (e, 0)),
            pl.BlockSpec((1, D, H), lambda e: (e, 0, 0)),
            pl.BlockSpec((1, 1, H), lambda e: (e, 0, 0)),
            pl.BlockSpec((1, H, O), lambda e: (e, 0, 0)),
            pl.BlockSpec((1, 1, O), lambda e: (e, 0, 0)),
            pl.BlockSpec((1, 1, capacity), lambda e: (e, 0, 0)),
            pl.BlockSpec((1, 1, capacity), lambda e: (e, 0, 0)),
        ],
        out_specs=pl.BlockSpec((T, O), lambda e: (0, 0)),
        out_shape=jax.ShapeDtypeStruct((T, O), jnp.float32),
        compiler_params=pltpu.CompilerParams(
            dimension_semantics=("arbitrary",),
        ),
    )(xg, w1, b1.reshape(E, 1, H), w2, b2.reshape(E, 1, O),
      tok_idx[:n_rows].reshape(E, 1, capacity),
      w_pos[:n_rows].reshape(E, 1, capacity))

    return out.reshape(B, S, O)


def kernel(x, Wr, w1, b1, w2, b2):
    return _moe(x, Wr, w1, b1, w2, b2)


# speedup vs baseline: 2.1155x; 1.0732x over previous
import functools

import jax
import jax.numpy as jnp
from jax import lax
from jax.experimental import pallas as pl
from jax.experimental.pallas import tpu as pltpu
from jax.experimental.pallas import tpu_sc as plsc

E = 8
K = 2
CAP_FACTOR = 1.5

_NC = 2
_NS = 16
_NW = _NC * _NS


def _router_kernel(x_ref, wr_ref, d_ref, c_ref, *, capacity, trash):
    x = x_ref[...]
    wr = wr_ref[...]
    T = x.shape[0]
    lt = jax.lax.dot_general(
        wr, x, (((1,), (1,)), ((), ())),
        preferred_element_type=jnp.float32)
    m = jnp.max(lt, axis=0, keepdims=True)
    ex = jnp.exp(lt - m)
    probs = ex / jnp.sum(ex, axis=0, keepdims=True)

    erow = jax.lax.broadcasted_iota(jnp.int32, probs.shape, 0)
    p1 = jnp.max(probs, axis=0, keepdims=True)
    a1 = jnp.min(jnp.where(probs == p1, erow, E), axis=0, keepdims=True)
    oh1 = (erow == a1).astype(jnp.float32)
    probs2 = jnp.where(erow == a1, -jnp.inf, probs)
    p2 = jnp.max(probs2, axis=0, keepdims=True)
    a2 = jnp.min(jnp.where(probs2 == p2, erow, E), axis=0, keepdims=True)
    oh2 = (erow == a2).astype(jnp.float32)

    s = p1 + p2
    p1n = p1 / s
    p2n = p2 / s

    ohsum = oh1 + oh2
    inc = ohsum
    shift = 1
    while shift < T:
        shifted = jnp.concatenate(
            [jnp.zeros((E, shift), jnp.float32), inc[:, : T - shift]], axis=1)
        inc = inc + shifted
        shift *= 2
    excl = inc - ohsum

    r1 = jnp.sum(oh1 * excl, axis=0, keepdims=True).astype(jnp.int32)
    r2 = jnp.sum(oh2 * excl, axis=0, keepdims=True).astype(jnp.int32)

    keep1 = r1 < capacity
    keep2 = r2 < capacity

    d1 = jnp.where(keep1, a1 * capacity + r1, trash)
    d2 = jnp.where(keep2, a2 * capacity + r2, trash)
    c1 = jnp.where(keep1, p1n, 0.0)
    c2 = jnp.where(keep2, p2n, 0.0)

    d_ref[...] = jnp.concatenate([d1, d2], axis=0)
    c_ref[...] = jnp.concatenate([c1, c2], axis=0)


def _sc_dispatch_gather_body(x_hbm, d_hbm, c_hbm, xg_hbm, tok_hbm, w_hbm,
                             dv_v, cv_v, tok_v, w_v, idx_v, rows_v, tok_sh,
                             sem, *, n_slots, n_tok, rows_per_w):
    cid = lax.axis_index("c")
    sid = lax.axis_index("s")

    @pl.when(sid == 0)
    def _():
        zi = jnp.zeros((16,), jnp.int32)
        zf = jnp.zeros((16,), jnp.float32)

        def zero_body(i, _):
            tok_v[pl.ds(i * 16, 16)] = zi
            w_v[pl.ds(i * 16, 16)] = zf
            return 0

        lax.fori_loop(0, n_slots // 16, zero_body, 0)

        iota16 = lax.iota(jnp.int32, 16)

        def scatter_from(k):
            pltpu.sync_copy(d_hbm.at[k], dv_v)
            pltpu.sync_copy(c_hbm.at[k], cv_v)

            def body(i, _):
                dv = dv_v[pl.ds(i * 16, 16)]
                cv = cv_v[pl.ds(i * 16, 16)]
                tokv = iota16 + i * 16
                plsc.store_scatter(tok_v, [dv], tokv)
                plsc.store_scatter(w_v, [dv], cv)
                return 0

            lax.fori_loop(0, n_tok // 16, body, 0)

        scatter_from(0)
        scatter_from(1)

        pltpu.sync_copy(tok_v, tok_sh)

        @pl.when(cid == 0)
        def _():
            pltpu.sync_copy(tok_v, tok_hbm)
            pltpu.sync_copy(w_v, w_hbm)

    plsc.subcore_barrier()

    base = (cid * _NS + sid) * rows_per_w
    pltpu.sync_copy(tok_sh.at[pl.ds(base, rows_per_w)], idx_v)
    pltpu.async_copy(x_hbm.at[idx_v], rows_v, sem).wait()
    pltpu.sync_copy(rows_v, xg_hbm.at[pl.ds(base, rows_per_w)])


def _ffn_combine_kernel(xg_ref, w1_ref, b1_ref, w2_ref, b2_ref, tok_ref,
                        w_ref, out_ref, y2_ref, *, capacity, eg):
    e = pl.program_id(0)
    for j in range(eg):
        xg = xg_ref[pl.ds(j * capacity, capacity), :]
        h = jnp.dot(xg, w1_ref[j], preferred_element_type=jnp.float32)
        h = h + b1_ref[j]
        h = 0.5 * h * (1.0 + jax.lax.erf(h * 0.7071067811865476))
        y = jnp.dot(h, w2_ref[j], preferred_element_type=jnp.float32)
        y2_ref[pl.ds(j * capacity, capacity), :] = y + b2_ref[j]

    T = out_ref.shape[0]
    tok_row = jnp.concatenate([tok_ref[j] for j in range(eg)], axis=1)
    w_row = jnp.concatenate([w_ref[j] for j in range(eg)], axis=1)
    t_col = jax.lax.broadcasted_iota(jnp.int32, (T, eg * capacity), 0)
    g = jnp.where(tok_row == t_col, w_row, 0.0)
    contrib = jnp.dot(g, y2_ref[...], preferred_element_type=jnp.float32)

    @pl.when(e == 0)
    def _():
        out_ref[...] = contrib

    @pl.when(e != 0)
    def _():
        out_ref[...] = out_ref[...] + contrib


@jax.jit
def _moe(x, Wr, w1, b1, w2, b2):
    B, S, D = x.shape
    T = B * S
    H = w1.shape[-1]
    O = w2.shape[-1]
    capacity = int((T / E) * CAP_FACTOR)
    n_rows = E * capacity
    n_slots = n_rows + 16
    trash = n_rows

    xt = x.reshape(T, D)

    d, c = pl.pallas_call(
        functools.partial(_router_kernel, capacity=capacity, trash=trash),
        out_shape=(
            jax.ShapeDtypeStruct((2, T), jnp.int32),
            jax.ShapeDtypeStruct((2, T), jnp.float32),
        ),
    )(xt, Wr)

    mesh = plsc.VectorSubcoreMesh(core_axis_name="c", subcore_axis_name="s")
    sc_params = pltpu.CompilerParams(needs_layout_passes=False)

    rows_per_w = n_rows // _NW
    xg, tok_idx, w_pos = pl.kernel(
        functools.partial(_sc_dispatch_gather_body, n_slots=n_slots,
                          n_tok=T, rows_per_w=rows_per_w),
        mesh=mesh,
        out_type=(
            jax.ShapeDtypeStruct((n_rows, D), jnp.float32),
            jax.ShapeDtypeStruct((n_slots,), jnp.int32),
            jax.ShapeDtypeStruct((n_slots,), jnp.float32),
        ),
        scratch_types=[
            pltpu.VMEM((T,), jnp.int32),
            pltpu.VMEM((T,), jnp.float32),
            pltpu.VMEM((n_slots,), jnp.int32),
            pltpu.VMEM((n_slots,), jnp.float32),
            pltpu.VMEM((rows_per_w,), jnp.int32),
            pltpu.VMEM((rows_per_w, D), jnp.float32),
            pltpu.VMEM_SHARED((n_slots,), jnp.int32),
            pltpu.SemaphoreType.DMA,
        ],
        compiler_params=sc_params,
    )(xt, d, c)

    eg = 2
    out = pl.pallas_call(
        functools.partial(_ffn_combine_kernel, capacity=capacity, eg=eg),
        grid=(E // eg,),
        in_specs=[
            pl.BlockSpec((eg * capacity, D), lambda e: (e, 0)),
            pl.BlockSpec((eg, D, H), lambda e: (e, 0, 0)),
            pl.BlockSpec((eg, 1, H), lambda e: (e, 0, 0)),
            pl.BlockSpec((eg, H, O), lambda e: (e, 0, 0)),
            pl.BlockSpec((eg, 1, O), lambda e: (e, 0, 0)),
            pl.BlockSpec((eg, 1, capacity), lambda e: (e, 0, 0)),
            pl.BlockSpec((eg, 1, capacity), lambda e: (e, 0, 0)),
        ],
        out_specs=pl.BlockSpec((T, O), lambda e: (0, 0)),
        out_shape=jax.ShapeDtypeStruct((T, O), jnp.float32),
        scratch_shapes=[pltpu.VMEM((eg * capacity, O), jnp.float32)],
        compiler_params=pltpu.CompilerParams(
            dimension_semantics=("arbitrary",),
        ),
    )(xg, w1, b1.reshape(E, 1, H), w2, b2.reshape(E, 1, O),
      tok_idx[:n_rows].reshape(E, 1, capacity),
      w_pos[:n_rows].reshape(E, 1, capacity))

    return out.reshape(B, S, O)


def kernel(x, Wr, w1, b1, w2, b2):
    return _moe(x, Wr, w1, b1, w2, b2)


# speedup vs baseline: 2.2055x; 1.0425x over previous
import functools

import jax
import jax.numpy as jnp
from jax import lax
from jax.experimental import pallas as pl
from jax.experimental.pallas import tpu as pltpu
from jax.experimental.pallas import tpu_sc as plsc

E = 8
K = 2
CAP_FACTOR = 1.5

_NC = 2
_NS = 16
_NW = _NC * _NS


def _router_kernel(x_ref, wr_ref, d_ref, c_ref, *, capacity, trash):
    x = x_ref[...]
    wr = wr_ref[...]
    T = x.shape[0]
    lt = jax.lax.dot_general(
        wr, x, (((1,), (1,)), ((), ())),
        preferred_element_type=jnp.float32)
    m = jnp.max(lt, axis=0, keepdims=True)
    ex = jnp.exp(lt - m)
    probs = ex / jnp.sum(ex, axis=0, keepdims=True)

    erow = jax.lax.broadcasted_iota(jnp.int32, probs.shape, 0)
    p1 = jnp.max(probs, axis=0, keepdims=True)
    a1 = jnp.min(jnp.where(probs == p1, erow, E), axis=0, keepdims=True)
    oh1 = (erow == a1).astype(jnp.float32)
    probs2 = jnp.where(erow == a1, -jnp.inf, probs)
    p2 = jnp.max(probs2, axis=0, keepdims=True)
    a2 = jnp.min(jnp.where(probs2 == p2, erow, E), axis=0, keepdims=True)
    oh2 = (erow == a2).astype(jnp.float32)

    s = p1 + p2
    p1n = p1 / s
    p2n = p2 / s

    ohsum = oh1 + oh2
    inc = ohsum
    shift = 1
    while shift < T:
        shifted = jnp.concatenate(
            [jnp.zeros((E, shift), jnp.float32), inc[:, : T - shift]], axis=1)
        inc = inc + shifted
        shift *= 2
    excl = inc - ohsum

    r1 = jnp.sum(oh1 * excl, axis=0, keepdims=True).astype(jnp.int32)
    r2 = jnp.sum(oh2 * excl, axis=0, keepdims=True).astype(jnp.int32)

    keep1 = r1 < capacity
    keep2 = r2 < capacity

    d1 = jnp.where(keep1, a1 * capacity + r1, trash)
    d2 = jnp.where(keep2, a2 * capacity + r2, trash)
    c1 = jnp.where(keep1, p1n, 0.0)
    c2 = jnp.where(keep2, p2n, 0.0)

    d_ref[...] = jnp.concatenate([d1, d2], axis=0)
    c_ref[...] = jnp.concatenate([c1, c2], axis=0)


def _sc_dispatch_gather_body(x_hbm, d_hbm, c_hbm, xg_hbm, tok_hbm, w_hbm,
                             dv_v, cv_v, tok_v, w_v, idx_v, rows_v, tok_sh,
                             sem, *, n_slots, n_tok, rows_per_w):
    cid = lax.axis_index("c")
    sid = lax.axis_index("s")

    iota16 = lax.iota(jnp.int32, 16)
    tmask = n_tok - 1
    zi = jnp.zeros((16,), jnp.int32)
    zf = jnp.zeros((16,), jnp.float32)

    @pl.when(sid == 0)
    def _():
        cpd0 = pltpu.async_copy(d_hbm.at[0], dv_v.at[pl.ds(0, n_tok)], sem)
        cpd1 = pltpu.async_copy(d_hbm.at[1], dv_v.at[pl.ds(n_tok, n_tok)], sem)

        def zero_body(i, _):
            tok_v[pl.ds(i * 16, 16)] = zi
            return 0

        lax.fori_loop(0, n_slots // 16, zero_body, 0)
        cpd0.wait()
        cpd1.wait()

        def tok_body(i, _):
            dv = dv_v[pl.ds(i * 16, 16)]
            tokv = (iota16 + i * 16) & tmask
            plsc.store_scatter(tok_v, [dv], tokv)
            return 0

        lax.fori_loop(0, (2 * n_tok) // 16, tok_body, 0)
        pltpu.sync_copy(tok_v, tok_sh)

        @pl.when(cid == 0)
        def _():
            pltpu.sync_copy(tok_v, tok_hbm)

    @pl.when(jnp.logical_and(sid == 1, cid == 0))
    def _():
        cpd0 = pltpu.async_copy(d_hbm.at[0], dv_v.at[pl.ds(0, n_tok)], sem)
        cpd1 = pltpu.async_copy(d_hbm.at[1], dv_v.at[pl.ds(n_tok, n_tok)], sem)
        cpc0 = pltpu.async_copy(c_hbm.at[0], cv_v.at[pl.ds(0, n_tok)], sem)
        cpc1 = pltpu.async_copy(c_hbm.at[1], cv_v.at[pl.ds(n_tok, n_tok)], sem)

        def zero_body(i, _):
            w_v[pl.ds(i * 16, 16)] = zf
            return 0

        lax.fori_loop(0, n_slots // 16, zero_body, 0)
        cpd0.wait()
        cpd1.wait()
        cpc0.wait()
        cpc1.wait()

        def w_body(i, _):
            dv = dv_v[pl.ds(i * 16, 16)]
            cv = cv_v[pl.ds(i * 16, 16)]
            plsc.store_scatter(w_v, [dv], cv)
            return 0

        lax.fori_loop(0, (2 * n_tok) // 16, w_body, 0)
        pltpu.sync_copy(w_v, w_hbm)

    plsc.subcore_barrier()

    base = (cid * _NS + sid) * rows_per_w
    pltpu.sync_copy(tok_sh.at[pl.ds(base, rows_per_w)], idx_v)
    pltpu.async_copy(x_hbm.at[idx_v], rows_v, sem).wait()
    pltpu.sync_copy(rows_v, xg_hbm.at[pl.ds(base, rows_per_w)])


def _ffn_combine_kernel(xg_ref, w1_ref, b1_ref, w2_ref, b2_ref, tok_ref,
                        w_ref, out_ref, y2_ref, *, capacity, eg):
    e = pl.program_id(0)
    for j in range(eg):
        xg = xg_ref[pl.ds(j * capacity, capacity), :]
        h = jnp.dot(xg, w1_ref[j], preferred_element_type=jnp.float32)
        h = h + b1_ref[j]
        h = 0.5 * h * (1.0 + jax.lax.erf(h * 0.7071067811865476))
        y = jnp.dot(h, w2_ref[j], preferred_element_type=jnp.float32)
        y2_ref[pl.ds(j * capacity, capacity), :] = y + b2_ref[j]

    T = out_ref.shape[0]
    tok_row = jnp.concatenate([tok_ref[j] for j in range(eg)], axis=1)
    w_row = jnp.concatenate([w_ref[j] for j in range(eg)], axis=1)
    t_col = jax.lax.broadcasted_iota(jnp.int32, (T, eg * capacity), 0)
    g = jnp.where(tok_row == t_col, w_row, 0.0)
    contrib = jnp.dot(g, y2_ref[...], preferred_element_type=jnp.float32)

    @pl.when(e == 0)
    def _():
        out_ref[...] = contrib

    @pl.when(e != 0)
    def _():
        out_ref[...] = out_ref[...] + contrib


@jax.jit
def _moe(x, Wr, w1, b1, w2, b2):
    B, S, D = x.shape
    T = B * S
    H = w1.shape[-1]
    O = w2.shape[-1]
    capacity = int((T / E) * CAP_FACTOR)
    n_rows = E * capacity
    n_slots = n_rows + 16
    trash = n_rows

    xt = x.reshape(T, D)

    d, c = pl.pallas_call(
        functools.partial(_router_kernel, capacity=capacity, trash=trash),
        out_shape=(
            jax.ShapeDtypeStruct((2, T), jnp.int32),
            jax.ShapeDtypeStruct((2, T), jnp.float32),
        ),
    )(xt, Wr)

    mesh = plsc.VectorSubcoreMesh(core_axis_name="c", subcore_axis_name="s")
    sc_params = pltpu.CompilerParams(needs_layout_passes=False)

    rows_per_w = n_rows // _NW
    xg, tok_idx, w_pos = pl.kernel(
        functools.partial(_sc_dispatch_gather_body, n_slots=n_slots,
                          n_tok=T, rows_per_w=rows_per_w),
        mesh=mesh,
        out_type=(
            jax.ShapeDtypeStruct((n_rows, D), jnp.float32),
            jax.ShapeDtypeStruct((n_slots,), jnp.int32),
            jax.ShapeDtypeStruct((n_slots,), jnp.float32),
        ),
        scratch_types=[
            pltpu.VMEM((2 * T,), jnp.int32),
            pltpu.VMEM((2 * T,), jnp.float32),
            pltpu.VMEM((n_slots,), jnp.int32),
            pltpu.VMEM((n_slots,), jnp.float32),
            pltpu.VMEM((rows_per_w,), jnp.int32),
            pltpu.VMEM((rows_per_w, D), jnp.float32),
            pltpu.VMEM_SHARED((n_slots,), jnp.int32),
            pltpu.SemaphoreType.DMA,
        ],
        compiler_params=sc_params,
    )(xt, d, c)

    eg = 2
    out = pl.pallas_call(
        functools.partial(_ffn_combine_kernel, capacity=capacity, eg=eg),
        grid=(E // eg,),
        in_specs=[
            pl.BlockSpec((eg * capacity, D), lambda e: (e, 0)),
            pl.BlockSpec((eg, D, H), lambda e: (e, 0, 0)),
            pl.BlockSpec((eg, 1, H), lambda e: (e, 0, 0)),
            pl.BlockSpec((eg, H, O), lambda e: (e, 0, 0)),
            pl.BlockSpec((eg, 1, O), lambda e: (e, 0, 0)),
            pl.BlockSpec((eg, 1, capacity), lambda e: (e, 0, 0)),
            pl.BlockSpec((eg, 1, capacity), lambda e: (e, 0, 0)),
        ],
        out_specs=pl.BlockSpec((T, O), lambda e: (0, 0)),
        out_shape=jax.ShapeDtypeStruct((T, O), jnp.float32),
        scratch_shapes=[pltpu.VMEM((eg * capacity, O), jnp.float32)],
        compiler_params=pltpu.CompilerParams(
            dimension_semantics=("arbitrary",),
        ),
    )(xg, w1, b1.reshape(E, 1, H), w2, b2.reshape(E, 1, O),
      tok_idx[:n_rows].reshape(E, 1, capacity),
      w_pos[:n_rows].reshape(E, 1, capacity))

    return out.reshape(B, S, O)


def kernel(x, Wr, w1, b1, w2, b2):
    return _moe(x, Wr, w1, b1, w2, b2)
